# baseline (speedup 1.0000x reference)
import functools

import jax
import jax.numpy as jnp
from jax import lax
from jax.experimental import pallas as pl
from jax.experimental.pallas import tpu as pltpu
from jax.experimental.pallas import tpu_sc as plsc

NC = 2
NS = 16
EC = 128
NB = 2
NP = 8
CHP = 80
BLK = 400
EPS_DEG = 1e-12
EPS_BN = 1e-5


def _mesh():
    return plsc.VectorSubcoreMesh(core_axis_name="c", subcore_axis_name="s")


def _zero_vmem_rows(ref, nrows, d):
    def row(i, _):
        for j in range(d // 16):
            ref[i, pl.ds(j * 16, 16)] = jnp.zeros((16,), jnp.float32)
        return 0
    lax.fori_loop(0, nrows, row, 0)


def _zero_vmem_1d(ref, n):
    def blk(i, _):
        ref[pl.ds(i * 16, 16)] = jnp.zeros((16,), jnp.float32)
        return 0
    lax.fori_loop(0, n // 16, blk, 0)


def _sc_stats(dst3, seg3, n, g):
    nw, n_ch, ec = dst3.shape
    assert ec == EC and nw == NC * NS
    ns_seg, nb_seg, _ = seg3.shape
    na = n + NP
    gp = g + NP
    n_grp, n_tail = n_ch // 4, n_ch % 4

    @functools.partial(
        pl.kernel,
        out_type=(jax.ShapeDtypeStruct((NC * n,), jnp.float32),
                  jax.ShapeDtypeStruct((NC * g,), jnp.float32)),
        mesh=_mesh(),
        scratch_types=[
            pltpu.VMEM_SHARED((na,), jnp.float32),
            pltpu.VMEM_SHARED((gp,), jnp.float32),
            pltpu.VMEM((128,), jnp.float32),
            pltpu.VMEM((EC,), jnp.float32),
            pltpu.VMEM((n_ch, EC), jnp.int32),
            pltpu.VMEM((nb_seg, EC), jnp.int32),
        ] + [pltpu.SemaphoreType.DMA for _ in range(4)],
    )
    def k(dst_hbm, seg_hbm, degp, cntp, deg_acc, cnt_acc, zbuf, ones,
          didx, bidx, *sems):
        c = lax.axis_index("c")
        s = lax.axis_index("s")
        w = c * NS + s
        pltpu.sync_copy(dst_hbm.at[w], didx)
        @pl.when(c == 0)
        def _():
            pltpu.sync_copy(seg_hbm.at[s], bidx)
        _zero_vmem_1d(zbuf, 128)
        def one_blk(i, _):
            ones[pl.ds(i * 16, 16)] = jnp.ones((16,), jnp.float32)
            return 0
        lax.fori_loop(0, EC // 16, one_blk, 0)
        nz_full, nz_tail = na // 128, na % 128
        for kk in range((nz_full + NS - 1) // NS):
            zc = s + NS * kk
            @pl.when(zc < nz_full)
            def _():
                off = pl.multiple_of(zc * 128, 8)
                pltpu.sync_copy(zbuf, deg_acc.at[pl.ds(off, 128)])
        if nz_tail:
            @pl.when(s == 0)
            def _():
                pltpu.sync_copy(zbuf.at[pl.ds(0, nz_tail)],
                                deg_acc.at[pl.ds(nz_full * 128, nz_tail)])
        @pl.when(s == 1)
        def _():
            for q in range(gp // 128):
                pltpu.sync_copy(zbuf, cnt_acc.at[pl.ds(q * 128, 128)])
            rem = gp % 128
            if rem:
                pltpu.sync_copy(zbuf.at[pl.ds(0, rem)],
                                cnt_acc.at[pl.ds(gp - rem, rem)])
        plsc.subcore_barrier()

        def grp(j, _):
            ds_ = [pltpu.async_copy(ones, deg_acc.at[didx.at[j * 4 + b]],
                                    sems[b], add=True) for b in range(4)]
            for b in range(4):
                ds_[b].wait()
            return 0
        lax.fori_loop(0, n_grp, grp, 0)
        for t in range(n_tail):
            pltpu.async_copy(ones, deg_acc.at[didx.at[n_grp * 4 + t]],
                             sems[0], add=True).wait()

        @pl.when(c == 0)
        def _():
            for j in range(nb_seg):
                pltpu.async_copy(ones, cnt_acc.at[bidx.at[j]],
                                 sems[j % 4], add=True).wait()
        plsc.subcore_barrier()

        nch = n // EC
        for kk in range((nch + NS - 1) // NS):
            ch = s + NS * kk
            @pl.when(ch < nch)
            def _():
                off = pl.multiple_of(ch * EC, 8)
                pltpu.sync_copy(deg_acc.at[pl.ds(off, EC)], ones)
                pltpu.sync_copy(ones, degp.at[pl.ds(c * n + off, EC)])
        rem_n = n - nch * EC
        if rem_n:
            @pl.when(s == NS - 1)
            def _():
                off = nch * EC
                pltpu.sync_copy(deg_acc.at[pl.ds(off, rem_n)],
                                ones.at[pl.ds(0, rem_n)])
                pltpu.sync_copy(ones.at[pl.ds(0, rem_n)],
                                degp.at[pl.ds(c * n + off, rem_n)])
        for kk in range(g // EC):
            @pl.when(s == kk)
            def _():
                pltpu.sync_copy(cnt_acc.at[pl.ds(kk * EC, EC)], ones)
                pltpu.sync_copy(ones, cntp.at[pl.ds(c * g + kk * EC, EC)])

    return k(dst3, seg3)


def _sc_agg(hs, src_flat, dst_flat, n_ch):
    n, d = hs.shape
    na = n + NP
    ept = n_ch * EC
    n_grp, n_tail = n_ch // NB, n_ch % NB
    nzf, nzt = n // EC, n % EC
    naf, nat = na // EC, na % EC

    @functools.partial(
        pl.kernel,
        out_type=jax.ShapeDtypeStruct((NC, n, d), jnp.float32),
        mesh=_mesh(),
        scratch_types=[
            pltpu.VMEM_SHARED((na, d), jnp.float32),
            pltpu.VMEM((ept,), jnp.int32),
        ] + [pltpu.VMEM((1, EC), jnp.int32) for _ in range(NB)]
          + [pltpu.VMEM((EC, d), jnp.float32) for _ in range(NB)]
          + [pltpu.SemaphoreType.DMA for _ in range(3 * NB)],
    )
    def k(hs_hbm, src_hbm, dst_hbm, out, acc, sidx, *rest):
        dbs = rest[:NB]
        bufs = rest[NB:2 * NB]
        dss = rest[2 * NB:3 * NB]
        gss = rest[3 * NB:4 * NB]
        sss = rest[4 * NB:5 * NB]
        buf0 = bufs[0]
        c = lax.axis_index("c")
        s = lax.axis_index("s")
        w = c * NS + s
        eoff = w * ept
        pltpu.sync_copy(src_hbm.at[pl.ds(eoff, ept)], sidx)

        _zero_vmem_rows(buf0, EC, d)
        for kk in range((naf + NS - 1) // NS):
            zc = s + NS * kk
            @pl.when(zc < naf)
            def _():
                off = pl.multiple_of(zc * EC, 8)
                pltpu.sync_copy(buf0, acc.at[pl.ds(off, EC)])
        if nat:
            @pl.when(s == 0)
            def _():
                pltpu.sync_copy(buf0.at[pl.ds(0, nat)],
                                acc.at[pl.ds(naf * EC, nat)])
        plsc.subcore_barrier()

        def grp(j, _):
            es = [pl.multiple_of((NB * j + b) * EC, 8) for b in range(NB)]
            ds_ = [pltpu.async_copy(dst_hbm.at[pl.ds(eoff + es[b], EC)],
                                    dbs[b].at[0], dss[b]) for b in range(NB)]
            gs_ = [pltpu.async_copy(hs_hbm.at[sidx.at[pl.ds(es[b], EC)]],
                                    bufs[b], gss[b]) for b in range(NB)]
            ss_ = []
            for b in range(NB):
                gs_[b].wait()
                ds_[b].wait()
                ss_.append(pltpu.async_copy(bufs[b], acc.at[dbs[b].at[0]],
                                            sss[b], add=True))
            for b in range(NB):
                ss_[b].wait()
            return 0
        lax.fori_loop(0, n_grp, grp, 0)
        for t in range(n_tail):
            tc = n_grp * NB + t
            et = pl.multiple_of(tc * EC, 8)
            pltpu.async_copy(dst_hbm.at[pl.ds(eoff + et, EC)],
                             dbs[0].at[0], dss[0]).wait()
            pltpu.async_copy(hs_hbm.at[sidx.at[pl.ds(et, EC)]], buf0,
                             gss[0]).wait()
            pltpu.async_copy(buf0, acc.at[dbs[0].at[0]], sss[0],
                             add=True).wait()
        plsc.subcore_barrier()

        for kk in range((nzf + NS - 1) // NS):
            zc = s + NS * kk
            @pl.when(zc < nzf)
            def _():
                off = pl.multiple_of(zc * EC, 8)
                pltpu.sync_copy(acc.at[pl.ds(off, EC)], buf0)
                pltpu.sync_copy(buf0, out.at[c, pl.ds(off, EC)])
        if nzt:
            @pl.when(s == 0)
            def _():
                off = nzf * EC
                pltpu.sync_copy(acc.at[pl.ds(off, nzt)],
                                buf0.at[pl.ds(0, nzt)])
                pltpu.sync_copy(buf0.at[pl.ds(0, nzt)],
                                out.at[c, pl.ds(off, nzt)])

    return k(hs, src_flat, dst_flat)


def _sc_pool(t, batch_seg, g):
    n, d = t.shape
    nb = n // CHP
    assert n % CHP == 0
    nw = NC * NS

    @functools.partial(
        pl.kernel,
        out_type=jax.ShapeDtypeStruct((NC, g, d), jnp.float32),
        mesh=_mesh(),
        scratch_types=[
            pltpu.VMEM_SHARED((g, d), jnp.float32),
            pltpu.VMEM((CHP, d), jnp.float32),
            pltpu.VMEM((CHP,), jnp.int32),
            pltpu.SemaphoreType.DMA,
        ],
    )
    def k(t_hbm, seg_hbm, out, acc, rows, bidx, sem):
        c = lax.axis_index("c")
        s = lax.axis_index("s")
        _zero_vmem_rows(rows, CHP, d)
        for kk in range((g + CHP - 1) // CHP):
            blkrows = min(CHP, g - kk * CHP)
            @pl.when(s == kk % NS)
            def _():
                pltpu.sync_copy(rows.at[pl.ds(0, blkrows)],
                                acc.at[pl.ds(kk * CHP, blkrows)])
        plsc.subcore_barrier()

        w = c * NS + s
        def body(kk, _):
            ch = w + nw * kk
            @pl.when(ch < nb)
            def _():
                off = pl.multiple_of(ch * CHP, 8)
                pltpu.sync_copy(t_hbm.at[pl.ds(off, CHP)], rows)
                pltpu.sync_copy(seg_hbm.at[pl.ds(off, CHP)], bidx)
                pltpu.sync_copy(rows, acc.at[bidx], add=True)
            return 0
        lax.fori_loop(0, (nb + nw - 1) // nw, body, 0)
        plsc.subcore_barrier()

        for kk in range((g + CHP - 1) // CHP):
            sz = min(CHP, g - kk * CHP)
            @pl.when(s == kk)
            def _():
                pltpu.sync_copy(acc.at[pl.ds(kk * CHP, sz)],
                                rows.at[pl.ds(0, sz)])
                pltpu.sync_copy(rows.at[pl.ds(0, sz)],
                                out.at[c, pl.ds(kk * CHP, sz)])

    return k(t, batch_seg)


def _tc_prep(degp_t, cntp_t):
    n = degp_t.shape[0]
    g = cntp_t.shape[0]

    def body(dp_ref, cp_ref, dinv_ref, winv_ref):
        deg = jnp.sum(dp_ref[...], axis=1, keepdims=True) + 1.0
        dinv_ref[...] = lax.rsqrt(jnp.maximum(deg, EPS_DEG))
        cnt = jnp.sum(cp_ref[...], axis=1, keepdims=True)
        winv_ref[...] = 1.0 / jnp.maximum(cnt, 1.0)

    return pl.pallas_call(
        body,
        out_shape=(jax.ShapeDtypeStruct((n, 1), jnp.float32),
                   jax.ShapeDtypeStruct((g, 1), jnp.float32)),
    )(degp_t, cntp_t)


def _bn_affine(stats, gamma, beta, n):
    mu = stats[0:1, :] / n
    var = stats[1:2, :] / n - mu * mu
    a = gamma * lax.rsqrt(var + EPS_BN)
    return a, beta - a * mu


def _tc_layer(t, stats, gamma, beta, w, dinv, first):
    n, d = t.shape

    def body(t_ref, st_ref, g_ref, be_ref, w_ref, di_ref, o_ref):
        x = t_ref[...]
        if not first:
            a, c = _bn_affine(st_ref[...], g_ref[...], be_ref[...], n)
            x = jnp.maximum(a * x + c, 0.0)
        h = jnp.dot(x, w_ref[...], preferred_element_type=jnp.float32)
        o_ref[...] = di_ref[...] * h

    grid = (n // BLK,)
    return pl.pallas_call(
        body,
        grid=grid,
        in_specs=[
            pl.BlockSpec((BLK, d), lambda i: (i, 0)),
            pl.BlockSpec((2, d), lambda i: (0, 0)),
            pl.BlockSpec((1, d), lambda i: (0, 0)),
            pl.BlockSpec((1, d), lambda i: (0, 0)),
            pl.BlockSpec((d, d), lambda i: (0, 0)),
            pl.BlockSpec((BLK, 1), lambda i: (i, 0)),
        ],
        out_specs=pl.BlockSpec((BLK, d), lambda i: (i, 0)),
        out_shape=jax.ShapeDtypeStruct((n, d), jnp.float32),
    )(t, stats, gamma, beta, w, dinv)


def _tc_combine(p, hs, dinv, b):
    n, d = hs.shape

    def body(p_ref, hs_ref, di_ref, b_ref, t_ref, st_ref):
        i = pl.program_id(0)
        t = di_ref[...] * (p_ref[0] + p_ref[1] + hs_ref[...]) + b_ref[...]
        t_ref[...] = t
        @pl.when(i == 0)
        def _():
            st_ref[...] = jnp.zeros_like(st_ref)
        st_ref[...] += jnp.concatenate(
            [jnp.sum(t, axis=0, keepdims=True),
             jnp.sum(t * t, axis=0, keepdims=True)], axis=0)

    grid = (n // BLK,)
    return pl.pallas_call(
        body,
        grid=grid,
        in_specs=[
            pl.BlockSpec((2, BLK, d), lambda i: (0, i, 0)),
            pl.BlockSpec((BLK, d), lambda i: (i, 0)),
            pl.BlockSpec((BLK, 1), lambda i: (i, 0)),
            pl.BlockSpec((1, d), lambda i: (0, 0)),
        ],
        out_specs=(pl.BlockSpec((BLK, d), lambda i: (i, 0)),
                   pl.BlockSpec((2, d), lambda i: (0, 0))),
        out_shape=(jax.ShapeDtypeStruct((n, d), jnp.float32),
                   jax.ShapeDtypeStruct((2, d), jnp.float32)),
    )(p, hs, dinv, b)


def _tc_out(poolp, winv, stats, gamma, beta, w, b, n):
    g, d = poolp.shape[1], poolp.shape[2]
    dt = w.shape[1]

    def body(p_ref, wi_ref, st_ref, g_ref, be_ref, w_ref, b_ref, o_ref):
        pm = wi_ref[...] * (p_ref[0] + p_ref[1])
        a, c = _bn_affine(st_ref[...], g_ref[...], be_ref[...], n)
        y = a * pm + c
        o = jnp.dot(y, w_ref[...], preferred_element_type=jnp.float32)
        o = o + b_ref[...]
        o_ref[...] = jnp.where(o >= 0, o, 0.1 * o)

    return pl.pallas_call(
        body,
        out_shape=jax.ShapeDtypeStruct((g, dt), jnp.float32),
    )(poolp, winv, stats, gamma, beta, w, b)


def kernel(x, edge_index, batch_seg, W1, b1, W2, b2, W3, b3,
           g1, be1, g2, be2, g3, be3, Wout, bout):
    n, d = x.shape
    g = 512
    t_out = Wout.shape[1]
    nw = NC * NS
    e = edge_index.shape[1]
    epw = e // nw
    epad = -epw % EC
    src2 = edge_index[0].reshape(nw, epw)
    dst2 = edge_index[1].reshape(nw, epw)
    src2 = jnp.pad(src2, ((0, 0), (0, epad)))
    pad_bins = n + (jnp.arange(epad, dtype=dst2.dtype) % NP)
    dst2 = jnp.concatenate(
        [dst2, jnp.broadcast_to(pad_bins, (nw, epad))], axis=1)
    src_flat = src2.reshape(-1)
    dst_flat = dst2.reshape(-1)
    dst3 = dst2.reshape(nw, -1, EC)
    n_ch = dst3.shape[1]
    npad = -n % (NS * EC)
    seg_pad = jnp.concatenate(
        [batch_seg, g + (jnp.arange(npad, dtype=batch_seg.dtype) % NP)])
    seg3 = seg_pad.reshape(NS, -1, EC)

    degp, cntp = _sc_stats(dst3, seg3, n, g)
    dinv, winv = _tc_prep(degp.reshape(NC, n).T, cntp.reshape(NC, g).T)

    def gcn(t, stats, gamma, beta, w, b, first=False):
        hs = _tc_layer(t, stats, gamma, beta, w, dinv, first)
        p = _sc_agg(hs, src_flat, dst_flat, n_ch)
        return _tc_combine(p, hs, dinv, b.reshape(1, d))

    zstats = jnp.zeros((2, d), jnp.float32)
    one_row = jnp.ones((1, d), jnp.float32)
    t1, st1 = gcn(x, zstats, one_row, one_row * 0, W1, b1, first=True)
    t2, st2 = gcn(t1, st1, g1.reshape(1, d), be1.reshape(1, d), W2, b2)
    t3, st3 = gcn(t2, st2, g2.reshape(1, d), be2.reshape(1, d), W3, b3)

    poolp = _sc_pool(t3, batch_seg, g)
    tp = 128
    w_pad = jnp.pad(Wout, ((0, 0), (0, tp - t_out)))
    b_pad = jnp.pad(bout, (0, tp - t_out)).reshape(1, tp)
    out = _tc_out(poolp, winv, st3, g3.reshape(1, d), be3.reshape(1, d),
                  w_pad, b_pad, n)
    return out[:, :t_out]


# speedup vs baseline: 1.0468x; 1.0468x over previous
import functools

import jax
import jax.numpy as jnp
from jax import lax
from jax.experimental import pallas as pl
from jax.experimental.pallas import tpu as pltpu
from jax.experimental.pallas import tpu_sc as plsc

NC = 2
NS = 16
EC = 128
NB = 2
NP = 8
CHP = 80
BLK = 1000
EPS_DEG = 1e-12
EPS_BN = 1e-5


def _mesh():
    return plsc.VectorSubcoreMesh(core_axis_name="c", subcore_axis_name="s")


def _zero_vmem_rows(ref, nrows, d):
    def row(i, _):
        for j in range(d // 16):
            ref[i, pl.ds(j * 16, 16)] = jnp.zeros((16,), jnp.float32)
        return 0
    lax.fori_loop(0, nrows, row, 0)


def _zero_vmem_1d(ref, n):
    def blk(i, _):
        ref[pl.ds(i * 16, 16)] = jnp.zeros((16,), jnp.float32)
        return 0
    lax.fori_loop(0, n // 16, blk, 0)


def _sc_stats(dst3, seg3, n, g):
    nw, n_ch, ec = dst3.shape
    assert ec == EC and nw == NC * NS
    ns_seg, nb_seg, _ = seg3.shape
    na = n + NP
    gp = g + NP
    n_grp, n_tail = n_ch // 4, n_ch % 4

    @functools.partial(
        pl.kernel,
        out_type=(jax.ShapeDtypeStruct((NC * n,), jnp.float32),
                  jax.ShapeDtypeStruct((NC * g,), jnp.float32)),
        mesh=_mesh(),
        scratch_types=[
            pltpu.VMEM_SHARED((na,), jnp.float32),
            pltpu.VMEM_SHARED((gp,), jnp.float32),
            pltpu.VMEM((128,), jnp.float32),
            pltpu.VMEM((EC,), jnp.float32),
            pltpu.VMEM((n_ch, EC), jnp.int32),
            pltpu.VMEM((nb_seg, EC), jnp.int32),
        ] + [pltpu.SemaphoreType.DMA for _ in range(4)],
    )
    def k(dst_hbm, seg_hbm, degp, cntp, deg_acc, cnt_acc, zbuf, ones,
          didx, bidx, *sems):
        c = lax.axis_index("c")
        s = lax.axis_index("s")
        w = c * NS + s
        pltpu.sync_copy(dst_hbm.at[w], didx)
        @pl.when(c == 0)
        def _():
            pltpu.sync_copy(seg_hbm.at[s], bidx)
        _zero_vmem_1d(zbuf, 128)
        def one_blk(i, _):
            ones[pl.ds(i * 16, 16)] = jnp.ones((16,), jnp.float32)
            return 0
        lax.fori_loop(0, EC // 16, one_blk, 0)
        nz_full, nz_tail = na // 128, na % 128
        for kk in range((nz_full + NS - 1) // NS):
            zc = s + NS * kk
            @pl.when(zc < nz_full)
            def _():
                off = pl.multiple_of(zc * 128, 8)
                pltpu.sync_copy(zbuf, deg_acc.at[pl.ds(off, 128)])
        if nz_tail:
            @pl.when(s == 0)
            def _():
                pltpu.sync_copy(zbuf.at[pl.ds(0, nz_tail)],
                                deg_acc.at[pl.ds(nz_full * 128, nz_tail)])
        @pl.when(s == 1)
        def _():
            for q in range(gp // 128):
                pltpu.sync_copy(zbuf, cnt_acc.at[pl.ds(q * 128, 128)])
            rem = gp % 128
            if rem:
                pltpu.sync_copy(zbuf.at[pl.ds(0, rem)],
                                cnt_acc.at[pl.ds(gp - rem, rem)])
        plsc.subcore_barrier()

        def grp(j, _):
            ds_ = [pltpu.async_copy(ones, deg_acc.at[didx.at[j * 4 + b]],
                                    sems[b], add=True) for b in range(4)]
            for b in range(4):
                ds_[b].wait()
            return 0
        lax.fori_loop(0, n_grp, grp, 0)
        for t in range(n_tail):
            pltpu.async_copy(ones, deg_acc.at[didx.at[n_grp * 4 + t]],
                             sems[0], add=True).wait()

        @pl.when(c == 0)
        def _():
            for j in range(nb_seg):
                pltpu.async_copy(ones, cnt_acc.at[bidx.at[j]],
                                 sems[j % 4], add=True).wait()
        plsc.subcore_barrier()

        nch = n // EC
        for kk in range((nch + NS - 1) // NS):
            ch = s + NS * kk
            @pl.when(ch < nch)
            def _():
                off = pl.multiple_of(ch * EC, 8)
                pltpu.sync_copy(deg_acc.at[pl.ds(off, EC)], ones)
                pltpu.sync_copy(ones, degp.at[pl.ds(c * n + off, EC)])
        rem_n = n - nch * EC
        if rem_n:
            @pl.when(s == NS - 1)
            def _():
                off = nch * EC
                pltpu.sync_copy(deg_acc.at[pl.ds(off, rem_n)],
                                ones.at[pl.ds(0, rem_n)])
                pltpu.sync_copy(ones.at[pl.ds(0, rem_n)],
                                degp.at[pl.ds(c * n + off, rem_n)])
        for kk in range(g // EC):
            @pl.when(s == kk)
            def _():
                pltpu.sync_copy(cnt_acc.at[pl.ds(kk * EC, EC)], ones)
                pltpu.sync_copy(ones, cntp.at[pl.ds(c * g + kk * EC, EC)])

    return k(dst3, seg3)


def _sc_agg(hs, src_flat, dst_flat, n_ch):
    n, d = hs.shape
    na = n + NP
    ept = n_ch * EC
    n_grp, n_tail = n_ch // NB, n_ch % NB
    nzf, nzt = n // EC, n % EC
    naf, nat = na // EC, na % EC

    @functools.partial(
        pl.kernel,
        out_type=jax.ShapeDtypeStruct((NC, n, d), jnp.float32),
        mesh=_mesh(),
        scratch_types=[
            pltpu.VMEM_SHARED((na, d), jnp.float32),
            pltpu.VMEM((ept,), jnp.int32),
        ] + [pltpu.VMEM((1, EC), jnp.int32) for _ in range(NB)]
          + [pltpu.VMEM((EC, d), jnp.float32) for _ in range(NB)]
          + [pltpu.SemaphoreType.DMA for _ in range(3 * NB)],
    )
    def k(hs_hbm, src_hbm, dst_hbm, out, acc, sidx, *rest):
        dbs = rest[:NB]
        bufs = rest[NB:2 * NB]
        dss = rest[2 * NB:3 * NB]
        gss = rest[3 * NB:4 * NB]
        sss = rest[4 * NB:5 * NB]
        buf0 = bufs[0]
        c = lax.axis_index("c")
        s = lax.axis_index("s")
        w = c * NS + s
        eoff = w * ept
        pltpu.sync_copy(src_hbm.at[pl.ds(eoff, ept)], sidx)

        _zero_vmem_rows(buf0, EC, d)
        for kk in range((naf + NS - 1) // NS):
            zc = s + NS * kk
            @pl.when(zc < naf)
            def _():
                off = pl.multiple_of(zc * EC, 8)
                pltpu.sync_copy(buf0, acc.at[pl.ds(off, EC)])
        if nat:
            @pl.when(s == 0)
            def _():
                pltpu.sync_copy(buf0.at[pl.ds(0, nat)],
                                acc.at[pl.ds(naf * EC, nat)])
        plsc.subcore_barrier()

        def grp(j, _):
            es = [pl.multiple_of((NB * j + b) * EC, 8) for b in range(NB)]
            ds_ = [pltpu.async_copy(dst_hbm.at[pl.ds(eoff + es[b], EC)],
                                    dbs[b].at[0], dss[b]) for b in range(NB)]
            gs_ = [pltpu.async_copy(hs_hbm.at[sidx.at[pl.ds(es[b], EC)]],
                                    bufs[b], gss[b]) for b in range(NB)]
            ss_ = []
            for b in range(NB):
                gs_[b].wait()
                ds_[b].wait()
                ss_.append(pltpu.async_copy(bufs[b], acc.at[dbs[b].at[0]],
                                            sss[b], add=True))
            for b in range(NB):
                ss_[b].wait()
            return 0
        lax.fori_loop(0, n_grp, grp, 0)
        for t in range(n_tail):
            tc = n_grp * NB + t
            et = pl.multiple_of(tc * EC, 8)
            pltpu.async_copy(dst_hbm.at[pl.ds(eoff + et, EC)],
                             dbs[0].at[0], dss[0]).wait()
            pltpu.async_copy(hs_hbm.at[sidx.at[pl.ds(et, EC)]], buf0,
                             gss[0]).wait()
            pltpu.async_copy(buf0, acc.at[dbs[0].at[0]], sss[0],
                             add=True).wait()
        plsc.subcore_barrier()

        for kk in range((nzf + NS - 1) // NS):
            zc = s + NS * kk
            @pl.when(zc < nzf)
            def _():
                off = pl.multiple_of(zc * EC, 8)
                pltpu.sync_copy(acc.at[pl.ds(off, EC)], buf0)
                pltpu.sync_copy(buf0, out.at[c, pl.ds(off, EC)])
        if nzt:
            @pl.when(s == 0)
            def _():
                off = nzf * EC
                pltpu.sync_copy(acc.at[pl.ds(off, nzt)],
                                buf0.at[pl.ds(0, nzt)])
                pltpu.sync_copy(buf0.at[pl.ds(0, nzt)],
                                out.at[c, pl.ds(off, nzt)])

    return k(hs, src_flat, dst_flat)


def _sc_pool(t, batch_seg, g):
    n, d = t.shape
    nb = n // CHP
    assert n % CHP == 0
    nw = NC * NS

    @functools.partial(
        pl.kernel,
        out_type=jax.ShapeDtypeStruct((NC, g, d), jnp.float32),
        mesh=_mesh(),
        scratch_types=[
            pltpu.VMEM_SHARED((g, d), jnp.float32),
            pltpu.VMEM((CHP, d), jnp.float32),
            pltpu.VMEM((CHP,), jnp.int32),
            pltpu.SemaphoreType.DMA,
        ],
    )
    def k(t_hbm, seg_hbm, out, acc, rows, bidx, sem):
        c = lax.axis_index("c")
        s = lax.axis_index("s")
        _zero_vmem_rows(rows, CHP, d)
        for kk in range((g + CHP - 1) // CHP):
            blkrows = min(CHP, g - kk * CHP)
            @pl.when(s == kk % NS)
            def _():
                pltpu.sync_copy(rows.at[pl.ds(0, blkrows)],
                                acc.at[pl.ds(kk * CHP, blkrows)])
        plsc.subcore_barrier()

        w = c * NS + s
        def body(kk, _):
            ch = w + nw * kk
            @pl.when(ch < nb)
            def _():
                off = pl.multiple_of(ch * CHP, 8)
                pltpu.sync_copy(t_hbm.at[pl.ds(off, CHP)], rows)
                pltpu.sync_copy(seg_hbm.at[pl.ds(off, CHP)], bidx)
                pltpu.sync_copy(rows, acc.at[bidx], add=True)
            return 0
        lax.fori_loop(0, (nb + nw - 1) // nw, body, 0)
        plsc.subcore_barrier()

        for kk in range((g + CHP - 1) // CHP):
            sz = min(CHP, g - kk * CHP)
            @pl.when(s == kk)
            def _():
                pltpu.sync_copy(acc.at[pl.ds(kk * CHP, sz)],
                                rows.at[pl.ds(0, sz)])
                pltpu.sync_copy(rows.at[pl.ds(0, sz)],
                                out.at[c, pl.ds(kk * CHP, sz)])

    return k(t, batch_seg)


def _tc_prep(degp_t, cntp_t):
    n = degp_t.shape[0]
    g = cntp_t.shape[0]

    def body(dp_ref, cp_ref, dinv_ref, winv_ref):
        deg = jnp.sum(dp_ref[...], axis=1, keepdims=True) + 1.0
        dinv_ref[...] = lax.rsqrt(jnp.maximum(deg, EPS_DEG))
        cnt = jnp.sum(cp_ref[...], axis=1, keepdims=True)
        winv_ref[...] = 1.0 / jnp.maximum(cnt, 1.0)

    return pl.pallas_call(
        body,
        out_shape=(jax.ShapeDtypeStruct((n, 1), jnp.float32),
                   jax.ShapeDtypeStruct((g, 1), jnp.float32)),
    )(degp_t, cntp_t)


def _bn_affine(stats, gamma, beta, n):
    mu = stats[0:1, :] / n
    var = stats[1:2, :] / n - mu * mu
    a = gamma * lax.rsqrt(var + EPS_BN)
    return a, beta - a * mu


def _tc_layer(t, stats, gamma, beta, w, dinv, first):
    n, d = t.shape

    def body(t_ref, st_ref, g_ref, be_ref, w_ref, di_ref, o_ref):
        x = t_ref[...]
        if not first:
            a, c = _bn_affine(st_ref[...], g_ref[...], be_ref[...], n)
            x = jnp.maximum(a * x + c, 0.0)
        h = jnp.dot(x, w_ref[...], preferred_element_type=jnp.float32)
        o_ref[...] = di_ref[...] * h

    grid = (n // BLK,)
    return pl.pallas_call(
        body,
        grid=grid,
        in_specs=[
            pl.BlockSpec((BLK, d), lambda i: (i, 0)),
            pl.BlockSpec((2, d), lambda i: (0, 0)),
            pl.BlockSpec((1, d), lambda i: (0, 0)),
            pl.BlockSpec((1, d), lambda i: (0, 0)),
            pl.BlockSpec((d, d), lambda i: (0, 0)),
            pl.BlockSpec((BLK, 1), lambda i: (i, 0)),
        ],
        out_specs=pl.BlockSpec((BLK, d), lambda i: (i, 0)),
        out_shape=jax.ShapeDtypeStruct((n, d), jnp.float32),
    )(t, stats, gamma, beta, w, dinv)


def _tc_combine(p, hs, dinv, b):
    n, d = hs.shape

    def body(p_ref, hs_ref, di_ref, b_ref, t_ref, st_ref):
        i = pl.program_id(0)
        t = di_ref[...] * (p_ref[0] + p_ref[1] + hs_ref[...]) + b_ref[...]
        t_ref[...] = t
        @pl.when(i == 0)
        def _():
            st_ref[...] = jnp.zeros_like(st_ref)
        st_ref[...] += jnp.concatenate(
            [jnp.sum(t, axis=0, keepdims=True),
             jnp.sum(t * t, axis=0, keepdims=True)], axis=0)

    grid = (n // BLK,)
    return pl.pallas_call(
        body,
        grid=grid,
        in_specs=[
            pl.BlockSpec((2, BLK, d), lambda i: (0, i, 0)),
            pl.BlockSpec((BLK, d), lambda i: (i, 0)),
            pl.BlockSpec((BLK, 1), lambda i: (i, 0)),
            pl.BlockSpec((1, d), lambda i: (0, 0)),
        ],
        out_specs=(pl.BlockSpec((BLK, d), lambda i: (i, 0)),
                   pl.BlockSpec((2, d), lambda i: (0, 0))),
        out_shape=(jax.ShapeDtypeStruct((n, d), jnp.float32),
                   jax.ShapeDtypeStruct((2, d), jnp.float32)),
    )(p, hs, dinv, b)


def _tc_out(poolp, winv, stats, gamma, beta, w, b, n):
    g, d = poolp.shape[1], poolp.shape[2]
    dt = w.shape[1]

    def body(p_ref, wi_ref, st_ref, g_ref, be_ref, w_ref, b_ref, o_ref):
        pm = wi_ref[...] * (p_ref[0] + p_ref[1])
        a, c = _bn_affine(st_ref[...], g_ref[...], be_ref[...], n)
        y = a * pm + c
        o = jnp.dot(y, w_ref[...], preferred_element_type=jnp.float32)
        o = o + b_ref[...]
        o_ref[...] = jnp.where(o >= 0, o, 0.1 * o)

    return pl.pallas_call(
        body,
        out_shape=jax.ShapeDtypeStruct((g, dt), jnp.float32),
    )(poolp, winv, stats, gamma, beta, w, b)


def kernel(x, edge_index, batch_seg, W1, b1, W2, b2, W3, b3,
           g1, be1, g2, be2, g3, be3, Wout, bout):
    n, d = x.shape
    g = 512
    t_out = Wout.shape[1]
    nw = NC * NS
    e = edge_index.shape[1]
    epw = e // nw
    epad = -epw % EC
    src2 = edge_index[0].reshape(nw, epw)
    dst2 = edge_index[1].reshape(nw, epw)
    src2 = jnp.pad(src2, ((0, 0), (0, epad)))
    pad_bins = n + (jnp.arange(epad, dtype=dst2.dtype) % NP)
    dst2 = jnp.concatenate(
        [dst2, jnp.broadcast_to(pad_bins, (nw, epad))], axis=1)
    src_flat = src2.reshape(-1)
    dst_flat = dst2.reshape(-1)
    dst3 = dst2.reshape(nw, -1, EC)
    n_ch = dst3.shape[1]
    npad = -n % (NS * EC)
    seg_pad = jnp.concatenate(
        [batch_seg, g + (jnp.arange(npad, dtype=batch_seg.dtype) % NP)])
    seg3 = seg_pad.reshape(NS, -1, EC)

    degp, cntp = _sc_stats(dst3, seg3, n, g)
    dinv, winv = _tc_prep(degp.reshape(NC, n).T, cntp.reshape(NC, g).T)

    def gcn(t, stats, gamma, beta, w, b, first=False):
        hs = _tc_layer(t, stats, gamma, beta, w, dinv, first)
        p = _sc_agg(hs, src_flat, dst_flat, n_ch)
        return _tc_combine(p, hs, dinv, b.reshape(1, d))

    zstats = jnp.zeros((2, d), jnp.float32)
    one_row = jnp.ones((1, d), jnp.float32)
    t1, st1 = gcn(x, zstats, one_row, one_row * 0, W1, b1, first=True)
    t2, st2 = gcn(t1, st1, g1.reshape(1, d), be1.reshape(1, d), W2, b2)
    t3, st3 = gcn(t2, st2, g2.reshape(1, d), be2.reshape(1, d), W3, b3)

    poolp = _sc_pool(t3, batch_seg, g)
    tp = 128
    w_pad = jnp.pad(Wout, ((0, 0), (0, tp - t_out)))
    b_pad = jnp.pad(bout, (0, tp - t_out)).reshape(1, tp)
    out = _tc_out(poolp, winv, st3, g3.reshape(1, d), be3.reshape(1, d),
                  w_pad, b_pad, n)
    return out[:, :t_out]
